# baseline (speedup 1.0000x reference)
import functools

import jax
import jax.numpy as jnp
from jax import lax
from jax.experimental import pallas as pl
from jax.experimental.pallas import tpu as pltpu
from jax.experimental.pallas import tpu_sc as plsc

VOCAB = 1000000
D_MODEL = 64
BATCH = 4096
SEQ = 200
NUM_CORES = 2
NUM_SUBCORES = 16
NW = NUM_CORES * NUM_SUBCORES
LANES = 128
JT = VOCAB // LANES
J_TAIL = VOCAB - JT * LANES
ROWS_P = VOCAB // 2
NJ = JT // NW + 1
NB = BATCH // LANES

_mesh = plsc.VectorSubcoreMesh(core_axis_name="c", subcore_axis_name="s")


def _iota16():
    return lax.broadcasted_iota(jnp.int32, (16,), 0)


def _wid():
    return lax.axis_index("s") * NUM_CORES + lax.axis_index("c")


@functools.partial(
    pl.kernel,
    mesh=_mesh,
    out_type=jax.ShapeDtypeStruct((ROWS_P, LANES), jnp.float32),
    scratch_types=[
        pltpu.VMEM((D_MODEL, LANES), jnp.float32),
        pltpu.VMEM((D_MODEL, LANES), jnp.float32),
        pltpu.VMEM((D_MODEL, J_TAIL), jnp.float32),
        pltpu.VMEM((J_TAIL // 2, LANES), jnp.float32),
    ],
    compiler_params=pltpu.CompilerParams(needs_layout_passes=False),
)
def _repack(wt_hbm, out_hbm, slab, outrow, slab_t, outrow_t):
    wid = _wid()

    def pair_rows(src, dst, v, _):
        for g in range(8):
            rows = _iota16() + 16 * (g % 4)
            col = jnp.zeros((16,), jnp.int32) + (2 * v + (1 if g >= 4 else 0))
            dst[v, pl.ds(16 * g, 16)] = plsc.load_gather(src, [rows, col])
        return _

    def body(n, carry):
        j = wid + n * NW

        @pl.when(j < JT)
        def _():
            pltpu.sync_copy(wt_hbm.at[:, pl.ds(j * LANES, LANES)], slab)
            lax.fori_loop(0, D_MODEL, functools.partial(pair_rows, slab, outrow), 0)
            pltpu.sync_copy(outrow, out_hbm.at[pl.ds(j * (LANES // 2), D_MODEL), :])

        return carry

    lax.fori_loop(0, NJ, body, 0)

    @pl.when(wid == 0)
    def _():
        pltpu.sync_copy(wt_hbm.at[:, pl.ds(JT * LANES, J_TAIL)], slab_t)
        lax.fori_loop(0, J_TAIL // 2,
                      functools.partial(pair_rows, slab_t, outrow_t), 0)
        pltpu.sync_copy(outrow_t, out_hbm.at[pl.ds(JT * (LANES // 2), J_TAIL // 2), :])


@functools.partial(
    pl.kernel,
    mesh=_mesh,
    out_type=jax.ShapeDtypeStruct((SEQ, D_MODEL, BATCH), jnp.float32),
    scratch_types=[
        pltpu.VMEM((SEQ, LANES), jnp.int32),
        pltpu.VMEM((SEQ, LANES), jnp.int32),
        pltpu.VMEM((LANES, LANES), jnp.float32),
        pltpu.VMEM((D_MODEL, LANES), jnp.float32),
        pltpu.SemaphoreType.DMA,
    ],
    compiler_params=pltpu.CompilerParams(needs_layout_passes=False),
)
def _gather(idst_hbm, table_hbm, out_hbm, idsb, qb, fetched, slab, gsem):
    wid = _wid()
    pltpu.sync_copy(idst_hbm.at[:, pl.ds(wid * LANES, LANES)], idsb)

    def make_q(t, carry):
        for g in range(8):
            qb[t, pl.ds(16 * g, 16)] = lax.shift_right_logical(
                idsb[t, pl.ds(16 * g, 16)], 1)
        return carry

    lax.fori_loop(0, SEQ, make_q, 0)

    def slab_body(t, carry):
        pltpu.async_copy(table_hbm.at[qb.at[t]], fetched, gsem).wait()

        def d_body(d, c):
            for g in range(8):
                rows = _iota16() + 16 * g
                ids16 = idsb[t, pl.ds(16 * g, 16)]
                off = lax.shift_left(jnp.bitwise_and(ids16, 1), 6)
                slab[d, pl.ds(16 * g, 16)] = plsc.load_gather(
                    fetched, [rows, off + d])
            return c

        lax.fori_loop(0, D_MODEL, d_body, 0)
        pltpu.sync_copy(slab, out_hbm.at[t, :, pl.ds(wid * LANES, LANES)])
        return carry

    lax.fori_loop(0, SEQ, slab_body, 0)


def kernel(input_ids, weight):
    tablep = _repack(weight.T)
    outt = _gather(input_ids.T, tablep)
    return outt.transpose(2, 0, 1)


# speedup vs baseline: 1.5337x; 1.5337x over previous
import functools

import jax
import jax.numpy as jnp
from jax import lax
from jax.experimental import pallas as pl
from jax.experimental.pallas import tpu as pltpu
from jax.experimental.pallas import tpu_sc as plsc

VOCAB = 1000000
D_MODEL = 64
BATCH = 4096
SEQ = 200
NUM_CORES = 2
NUM_SUBCORES = 16
NW = NUM_CORES * NUM_SUBCORES
LANES = 128
JT = VOCAB // LANES
J_TAIL = VOCAB - JT * LANES
ROWS_P = VOCAB // 2

_mesh = plsc.VectorSubcoreMesh(core_axis_name="c", subcore_axis_name="s")


def _iota16():
    return lax.broadcasted_iota(jnp.int32, (16,), 0)


def _cvec(val):
    return jnp.full((16,), val, jnp.int32)


def _wid():
    return lax.axis_index("s") * NUM_CORES + lax.axis_index("c")


def _pair_rows(src, dst, nv):
    for v in range(nv):
        for half in (0, 1):
            col = _cvec(2 * v + half)
            for gg in range(4):
                rows = _iota16() + 16 * gg
                dst[v, pl.ds(16 * (half * 4 + gg), 16)] = plsc.load_gather(
                    src, [rows, col])


@functools.partial(
    pl.kernel,
    mesh=_mesh,
    out_type=jax.ShapeDtypeStruct((ROWS_P, LANES), jnp.float32),
    scratch_types=[
        pltpu.VMEM((D_MODEL, LANES), jnp.float32),
        pltpu.VMEM((D_MODEL, LANES), jnp.float32),
        pltpu.VMEM((D_MODEL, LANES), jnp.float32),
        pltpu.VMEM((D_MODEL, LANES), jnp.float32),
        pltpu.VMEM((D_MODEL, J_TAIL), jnp.float32),
        pltpu.VMEM((J_TAIL // 2, LANES), jnp.float32),
        pltpu.SemaphoreType.DMA,
        pltpu.SemaphoreType.DMA,
        pltpu.SemaphoreType.DMA,
        pltpu.SemaphoreType.DMA,
    ],
    compiler_params=pltpu.CompilerParams(needs_layout_passes=False),
)
def _repack(wt_hbm, out_hbm, slab0, slab1, orow0, orow1, slab_t, orow_t,
            isem0, isem1, osem0, osem1):
    wid = _wid()
    slabs, orows = (slab0, slab1), (orow0, orow1)
    isems, osems = (isem0, isem1), (osem0, osem1)
    trip = jnp.where(wid < JT - (JT // NW) * NW, JT // NW + 1, JT // NW)

    def in_copy(n, b):
        j = wid + n * NW
        return pltpu.make_async_copy(
            wt_hbm.at[:, pl.ds(j * LANES, LANES)], slabs[b], isems[b])

    def out_copy(n, b):
        j = wid + n * NW
        return pltpu.make_async_copy(
            orows[b], out_hbm.at[pl.ds(j * (LANES // 2), D_MODEL), :], osems[b])

    for b in range(2):
        in_copy(b, b).start()

    n_groups = (JT // NW + 2) // 2

    def group(gi, carry):
        for b in range(2):
            n = gi * 2 + b

            @pl.when(n < trip)
            def _():
                in_copy(n, b).wait()

                @pl.when(n >= 2)
                def _():
                    out_copy(n - 2, b).wait()

                _pair_rows(slabs[b], orows[b], D_MODEL)
                out_copy(n, b).start()

                @pl.when(n + 2 < trip)
                def _():
                    in_copy(n + 2, b).start()

        return carry

    lax.fori_loop(0, n_groups, group, 0)

    for b in range(2):
        n_last = ((trip - 1 - b) // 2) * 2 + b
        out_copy(n_last, b).wait()

    @pl.when(wid == 0)
    def _():
        pltpu.sync_copy(wt_hbm.at[:, pl.ds(JT * LANES, J_TAIL)], slab_t)
        _pair_rows(slab_t, orow_t, J_TAIL // 2)
        pltpu.sync_copy(
            orow_t, out_hbm.at[pl.ds(JT * (LANES // 2), J_TAIL // 2), :])


@functools.partial(
    pl.kernel,
    mesh=_mesh,
    out_type=jax.ShapeDtypeStruct((SEQ, D_MODEL, BATCH), jnp.float32),
    scratch_types=[
        pltpu.VMEM((SEQ, LANES), jnp.int32),
        pltpu.VMEM((SEQ, LANES), jnp.int32),
        pltpu.VMEM((LANES, LANES), jnp.float32),
        pltpu.VMEM((LANES, LANES), jnp.float32),
        pltpu.VMEM((D_MODEL, LANES), jnp.float32),
        pltpu.VMEM((D_MODEL, LANES), jnp.float32),
        pltpu.SemaphoreType.DMA,
        pltpu.SemaphoreType.DMA,
        pltpu.SemaphoreType.DMA,
        pltpu.SemaphoreType.DMA,
    ],
    compiler_params=pltpu.CompilerParams(needs_layout_passes=False),
)
def _gather(idst_hbm, table_hbm, out_hbm, idsb, qb, fet0, fet1, slab0, slab1,
            gsem0, gsem1, wsem0, wsem1):
    wid = _wid()
    fets, slabs = (fet0, fet1), (slab0, slab1)
    gsems, wsems = (gsem0, gsem1), (wsem0, wsem1)
    pltpu.sync_copy(idst_hbm.at[:, pl.ds(wid * LANES, LANES)], idsb)

    def make_q(t, carry):
        for g in range(8):
            qb[t, pl.ds(16 * g, 16)] = lax.shift_right_logical(
                idsb[t, pl.ds(16 * g, 16)], 1)
        return carry

    lax.fori_loop(0, SEQ, make_q, 0)

    def g_copy(t, b):
        return pltpu.make_async_copy(table_hbm.at[qb.at[t]], fets[b], gsems[b])

    def w_copy(t, b):
        return pltpu.make_async_copy(
            slabs[b], out_hbm.at[t, :, pl.ds(wid * LANES, LANES)], wsems[b])

    def transpose_select(t, b):
        for g in range(8):
            rows = _iota16() + 16 * g
            ids16 = idsb[t, pl.ds(16 * g, 16)]
            off = lax.shift_left(jnp.bitwise_and(ids16, 1), 6)
            for d in range(D_MODEL):
                slabs[b][d, pl.ds(16 * g, 16)] = plsc.load_gather(
                    fets[b], [rows, off + d])

    for b in range(2):
        g_copy(b, b).start()

    def group(gi, carry):
        for b in range(2):
            t = gi * 2 + b
            g_copy(t, b).wait()

            @pl.when(t >= 2)
            def _():
                w_copy(t - 2, b).wait()

            transpose_select(t, b)
            w_copy(t, b).start()

            @pl.when(t + 2 < SEQ)
            def _():
                g_copy(t + 2, b).start()

        return carry

    lax.fori_loop(0, SEQ // 2, group, 0)
    for b in range(2):
        w_copy(SEQ - 2 + b, b).wait()


def kernel(input_ids, weight):
    tablep = _repack(weight.T)
    outt = _gather(input_ids.T, tablep)
    return outt.transpose(2, 0, 1)


# speedup vs baseline: 4.7711x; 3.1109x over previous
import functools

import jax
import jax.numpy as jnp
from jax import lax
from jax.experimental import pallas as pl
from jax.experimental.pallas import tpu as pltpu
from jax.experimental.pallas import tpu_sc as plsc

VOCAB = 1000000
D_MODEL = 64
BATCH = 4096
SEQ = 200
NUM_CORES = 2
NUM_SUBCORES = 16
NW = NUM_CORES * NUM_SUBCORES
LANES = 128
JT = VOCAB // LANES
J_TAIL = VOCAB - JT * LANES
ROWS_P = VOCAB // 2

_mesh = plsc.VectorSubcoreMesh(core_axis_name="c", subcore_axis_name="s")

def _wid():
    return lax.axis_index("s") * NUM_CORES + lax.axis_index("c")


def _skew_vecs():
    iota = lax.broadcasted_iota(jnp.int32, (16,), 0)
    pair_col = lax.shift_left(jnp.bitwise_and(iota, 1), 6)
    pair_row = lax.shift_right_logical(iota, 1)
    return iota, pair_col, pair_row


def _pair_rows(src, dst, nv, sv):
    iota, pair_col, pair_row = sv

    def kbody(k, carry):
        diag = jnp.bitwise_and(iota + k, 15)
        for c0 in range(0, 2 * nv, 16):
            for d0 in range(0, D_MODEL, 16):
                reg = plsc.load_gather(src, [d0 + diag, c0 + iota])
                plsc.store_scatter(
                    dst, [c0 // 2 + pair_row, d0 + pair_col + diag], reg)
        return carry

    lax.fori_loop(0, 16, kbody, 0)


@functools.partial(
    pl.kernel,
    mesh=_mesh,
    out_type=jax.ShapeDtypeStruct((ROWS_P, LANES), jnp.float32),
    scratch_types=[
        pltpu.VMEM((D_MODEL, LANES), jnp.float32),
        pltpu.VMEM((D_MODEL, LANES), jnp.float32),
        pltpu.VMEM((D_MODEL, LANES), jnp.float32),
        pltpu.VMEM((D_MODEL, LANES), jnp.float32),
        pltpu.VMEM((D_MODEL, J_TAIL), jnp.float32),
        pltpu.VMEM((J_TAIL // 2, LANES), jnp.float32),
        pltpu.SemaphoreType.DMA,
        pltpu.SemaphoreType.DMA,
        pltpu.SemaphoreType.DMA,
        pltpu.SemaphoreType.DMA,
    ],
    compiler_params=pltpu.CompilerParams(needs_layout_passes=False),
)
def _repack(wt_hbm, out_hbm, slab0, slab1, orow0, orow1, slab_t, orow_t,
            isem0, isem1, osem0, osem1):
    wid = _wid()
    sv = _skew_vecs()
    slabs, orows = (slab0, slab1), (orow0, orow1)
    isems, osems = (isem0, isem1), (osem0, osem1)
    trip = jnp.where(wid < JT - (JT // NW) * NW, JT // NW + 1, JT // NW)

    def in_copy(n, b):
        j = wid + n * NW
        return pltpu.make_async_copy(
            wt_hbm.at[:, pl.ds(j * LANES, LANES)], slabs[b], isems[b])

    def out_copy(n, b):
        j = wid + n * NW
        return pltpu.make_async_copy(
            orows[b], out_hbm.at[pl.ds(j * (LANES // 2), D_MODEL), :], osems[b])

    for b in range(2):
        in_copy(b, b).start()

    n_groups = (JT // NW + 2) // 2

    def group(gi, carry):
        for b in range(2):
            n = gi * 2 + b

            @pl.when(n < trip)
            def _():
                in_copy(n, b).wait()

                @pl.when(n >= 2)
                def _():
                    out_copy(n - 2, b).wait()

                _pair_rows(slabs[b], orows[b], D_MODEL, sv)
                out_copy(n, b).start()

                @pl.when(n + 2 < trip)
                def _():
                    in_copy(n + 2, b).start()

        return carry

    lax.fori_loop(0, n_groups, group, 0)

    for b in range(2):
        n_last = ((trip - 1 - b) // 2) * 2 + b
        out_copy(n_last, b).wait()

    @pl.when(wid == 0)
    def _():
        pltpu.sync_copy(wt_hbm.at[:, pl.ds(JT * LANES, J_TAIL)], slab_t)
        _pair_rows(slab_t, orow_t, J_TAIL // 2, sv)
        pltpu.sync_copy(
            orow_t, out_hbm.at[pl.ds(JT * (LANES // 2), J_TAIL // 2), :])


@functools.partial(
    pl.kernel,
    mesh=_mesh,
    out_type=jax.ShapeDtypeStruct((SEQ, D_MODEL, BATCH), jnp.float32),
    scratch_types=[
        pltpu.VMEM((SEQ, LANES), jnp.int32),
        pltpu.VMEM((SEQ, LANES), jnp.int32),
        pltpu.VMEM((LANES, LANES), jnp.float32),
        pltpu.VMEM((LANES, LANES), jnp.float32),
        pltpu.VMEM((D_MODEL, LANES), jnp.float32),
        pltpu.VMEM((D_MODEL, LANES), jnp.float32),
        pltpu.SemaphoreType.DMA,
        pltpu.SemaphoreType.DMA,
        pltpu.SemaphoreType.DMA,
        pltpu.SemaphoreType.DMA,
    ],
    compiler_params=pltpu.CompilerParams(needs_layout_passes=False),
)
def _gather(idst_hbm, table_hbm, out_hbm, idsb, qb, fet0, fet1, slab0, slab1,
            gsem0, gsem1, wsem0, wsem1):
    wid = _wid()
    iota, _, _ = _skew_vecs()
    fets, slabs = (fet0, fet1), (slab0, slab1)
    gsems, wsems = (gsem0, gsem1), (wsem0, wsem1)
    pltpu.sync_copy(idst_hbm.at[:, pl.ds(wid * LANES, LANES)], idsb)

    def make_q(t, carry):
        for g in range(8):
            qb[t, pl.ds(16 * g, 16)] = lax.shift_right_logical(
                idsb[t, pl.ds(16 * g, 16)], 1)
        return carry

    lax.fori_loop(0, SEQ, make_q, 0)

    def g_copy(t, b):
        return pltpu.make_async_copy(table_hbm.at[qb.at[t]], fets[b], gsems[b])

    def w_copy(t, b):
        return pltpu.make_async_copy(
            slabs[b], out_hbm.at[t, :, pl.ds(wid * LANES, LANES)], wsems[b])

    def transpose_select(t, b):
        offs = []
        for g in range(8):
            ids16 = idsb[t, pl.ds(16 * g, 16)]
            offs.append(lax.shift_left(jnp.bitwise_and(ids16, 1), 6))

        def kbody(k, carry):
            diag = jnp.bitwise_and(iota + k, 15)
            for g in range(8):
                base = 16 * g + iota
                for d0 in range(0, D_MODEL, 16):
                    reg = plsc.load_gather(
                        fets[b], [base, offs[g] + (d0 + diag)])
                    plsc.store_scatter(slabs[b], [d0 + diag, base], reg)
            return carry

        lax.fori_loop(0, 16, kbody, 0)

    for b in range(2):
        g_copy(b, b).start()

    def group(gi, carry):
        for b in range(2):
            t = gi * 2 + b
            g_copy(t, b).wait()

            @pl.when(t >= 2)
            def _():
                w_copy(t - 2, b).wait()

            transpose_select(t, b)
            w_copy(t, b).start()

            @pl.when(t + 2 < SEQ)
            def _():
                g_copy(t + 2, b).start()

        return carry

    lax.fori_loop(0, SEQ // 2, group, 0)
    for b in range(2):
        w_copy(SEQ - 2 + b, b).wait()


def kernel(input_ids, weight):
    tablep = _repack(weight.T)
    outt = _gather(input_ids.T, tablep)
    return outt.transpose(2, 0, 1)
